# baseline (speedup 1.0000x reference)
import jax
import jax.numpy as jnp
from jax.experimental import pallas as pl
from jax.experimental.pallas import tpu as pltpu

_B = 16
_T = 1024
_K = 1024
_D = 64
_COMMIT = 0.25


def _vq_body(z_ref, cb_ref, cbt_ref, qst_ref, idx_ref, part_ref):
    z2d = z_ref[0]
    cb = cb_ref[...]
    cbt = cbt_ref[...]
    a = jnp.sum(z2d * z2d, axis=1, keepdims=True)
    b = jax.lax.dot_general(z2d, cb, (((1,), (1,)), ((), ())),
                            preferred_element_type=jnp.float32)
    c = jnp.sum(cbt * cbt, axis=0, keepdims=True)
    dist = a - 2.0 * b + c
    min_d = jnp.min(dist, axis=1, keepdims=True)
    lane = jax.lax.broadcasted_iota(jnp.int32, (_T, _K), 1)
    idx2d = jnp.min(jnp.where(dist == min_d, lane, _K),
                    axis=1, keepdims=True)
    onehot = (lane == idx2d).astype(jnp.float32)
    q = jax.lax.dot_general(onehot, cb, (((1,), (0,)), ((), ())),
                            preferred_element_type=jnp.float32,
                            precision=jax.lax.Precision.HIGHEST)
    qst_ref[0] = z2d + (q - z2d)
    idx_ref[0] = idx2d
    part_ref[0] = jnp.sum(min_d, axis=(0, 1), keepdims=True)


def kernel(z, codebook):
    qst, idx3, parts = pl.pallas_call(
        _vq_body,
        grid=(_B,),
        in_specs=[
            pl.BlockSpec((1, _T, _D), lambda i: (i, 0, 0)),
            pl.BlockSpec((_K, _D), lambda i: (0, 0)),
            pl.BlockSpec((_D, _K), lambda i: (0, 0)),
        ],
        out_specs=[
            pl.BlockSpec((1, _T, _D), lambda i: (i, 0, 0)),
            pl.BlockSpec((1, _T, 1), lambda i: (i, 0, 0)),
            pl.BlockSpec((1, 1, 1), lambda i: (i, 0, 0)),
        ],
        out_shape=[
            jax.ShapeDtypeStruct((_B, _T, _D), jnp.float32),
            jax.ShapeDtypeStruct((_B, _T, 1), jnp.int32),
            jax.ShapeDtypeStruct((_B, 1, 1), jnp.float32),
        ],
        compiler_params=pltpu.CompilerParams(
            dimension_semantics=("parallel",)),
    )(z, codebook, codebook.T)
    vq_loss = jnp.sum(parts) * ((_COMMIT + 1.0) / z.size)
    return qst, idx3.reshape(z.shape[:-1]), vq_loss


# speedup vs baseline: 1.4175x; 1.4175x over previous
import jax
import jax.numpy as jnp
from jax.experimental import pallas as pl
from jax.experimental.pallas import tpu as pltpu

_B = 16
_T = 1024
_K = 1024
_D = 64
_COMMIT = 0.25


def _vq_body(z_ref, cb_ref, cbt_ref, qst_ref, idx_ref, part_ref):
    z2d = z_ref[0]
    cb = cb_ref[...]
    cbt = cbt_ref[...]
    a = jnp.sum(z2d * z2d, axis=1, keepdims=True)
    b = jax.lax.dot_general(z2d, cb, (((1,), (1,)), ((), ())),
                            preferred_element_type=jnp.float32)
    c = jnp.sum(cbt * cbt, axis=0, keepdims=True)
    dist = a - 2.0 * b + c
    min_d = jnp.min(dist, axis=1, keepdims=True)
    lane = jax.lax.broadcasted_iota(jnp.int32, (_T, _K), 1)
    idx2d = jnp.min(jnp.where(dist == min_d, lane, _K),
                    axis=1, keepdims=True)
    onehot = (lane == idx2d).astype(jnp.bfloat16)
    cb_hi = cb.astype(jnp.bfloat16)
    cb_lo = (cb - cb_hi.astype(jnp.float32)).astype(jnp.bfloat16)
    q = (jax.lax.dot_general(onehot, cb_hi, (((1,), (0,)), ((), ())),
                             preferred_element_type=jnp.float32)
         + jax.lax.dot_general(onehot, cb_lo, (((1,), (0,)), ((), ())),
                               preferred_element_type=jnp.float32))
    qst_ref[0] = z2d + (q - z2d)
    idx_ref[0] = idx2d
    part_ref[0] = jnp.sum(min_d, axis=(0, 1), keepdims=True)


def kernel(z, codebook):
    qst, idx3, parts = pl.pallas_call(
        _vq_body,
        grid=(_B,),
        in_specs=[
            pl.BlockSpec((1, _T, _D), lambda i: (i, 0, 0)),
            pl.BlockSpec((_K, _D), lambda i: (0, 0)),
            pl.BlockSpec((_D, _K), lambda i: (0, 0)),
        ],
        out_specs=[
            pl.BlockSpec((1, _T, _D), lambda i: (i, 0, 0)),
            pl.BlockSpec((1, _T, 1), lambda i: (i, 0, 0)),
            pl.BlockSpec((1, 1, 1), lambda i: (i, 0, 0)),
        ],
        out_shape=[
            jax.ShapeDtypeStruct((_B, _T, _D), jnp.float32),
            jax.ShapeDtypeStruct((_B, _T, 1), jnp.int32),
            jax.ShapeDtypeStruct((_B, 1, 1), jnp.float32),
        ],
        compiler_params=pltpu.CompilerParams(
            dimension_semantics=("parallel",)),
    )(z, codebook, codebook.T)
    vq_loss = jnp.sum(parts) * ((_COMMIT + 1.0) / z.size)
    return qst, idx3.reshape(z.shape[:-1]), vq_loss


# speedup vs baseline: 1.5799x; 1.1146x over previous
import functools

import jax
import jax.numpy as jnp
from jax.experimental import pallas as pl
from jax.experimental.pallas import tpu as pltpu
from jax.experimental.pallas import tpu_sc as plsc

_N = 16384
_T = 2048
_K = 1024
_D = 64
_COMMIT = 0.25

_SC_NC = 2
_SC_NS = 16
_NW = _SC_NC * _SC_NS
_BPW = _N // _NW


def _vq_body(z_ref, cbm2_ref, cbt_ref, idx_ref, part_ref):
    z2d = z_ref[...]
    cbm2 = cbm2_ref[...]
    cbt = cbt_ref[...]
    a = jnp.sum(z2d * z2d, axis=1, keepdims=True)
    b2 = jax.lax.dot_general(z2d, cbm2, (((1,), (1,)), ((), ())),
                             preferred_element_type=jnp.float32)
    c = jnp.sum(cbt * cbt, axis=0, keepdims=True)
    dist = a + b2 + c
    min_d = jnp.min(dist, axis=1, keepdims=True)
    lane = jax.lax.broadcasted_iota(jnp.int32, (_T, _K), 1)
    idx2d = jnp.min(jnp.where(dist == min_d, lane, _K),
                    axis=1, keepdims=True)
    idx_ref[...] = idx2d
    part_ref[0] = jnp.sum(min_d, axis=(0, 1), keepdims=True)


def _tc_argmin(zf, codebook):
    return pl.pallas_call(
        _vq_body,
        grid=(_N // _T,),
        in_specs=[
            pl.BlockSpec((_T, _D), lambda i: (i, 0)),
            pl.BlockSpec((_K, _D), lambda i: (0, 0)),
            pl.BlockSpec((_D, _K), lambda i: (0, 0)),
        ],
        out_specs=[
            pl.BlockSpec((_T, 1), lambda i: (i, 0)),
            pl.BlockSpec((1, 1, 1), lambda i: (i, 0, 0)),
        ],
        out_shape=[
            jax.ShapeDtypeStruct((_N, 1), jnp.int32),
            jax.ShapeDtypeStruct((_N // _T, 1, 1), jnp.float32),
        ],
        compiler_params=pltpu.CompilerParams(
            dimension_semantics=("parallel",)),
    )(zf, codebook * -2.0, codebook.T)


@functools.partial(
    pl.kernel,
    mesh=plsc.VectorSubcoreMesh(core_axis_name="c", subcore_axis_name="s"),
    out_type=jax.ShapeDtypeStruct((_N, 128), jnp.float32),
    scratch_types=[
        pltpu.VMEM((_BPW,), jnp.int32),
        pltpu.VMEM((_BPW, 128), jnp.float32),
        pltpu.SemaphoreType.DMA,
    ],
)
def _sc_gather(table_hbm, idx_hbm, out_hbm, idx_v, rows_v, sem):
    wid = jax.lax.axis_index("s") * _SC_NC + jax.lax.axis_index("c")
    base = wid * _BPW
    pltpu.sync_copy(idx_hbm.at[pl.ds(base, _BPW)], idx_v)
    pltpu.async_copy(table_hbm.at[idx_v], rows_v, sem).wait()
    pltpu.sync_copy(rows_v, out_hbm.at[pl.ds(base, _BPW)])


def kernel(z, codebook):
    zf = z.reshape(_N, _D)
    idx2, parts = _tc_argmin(zf, codebook)
    table_pad = jnp.concatenate(
        [codebook, jnp.zeros((_K, 128 - _D), jnp.float32)], axis=1)
    q = _sc_gather(table_pad, idx2.reshape(_N))[:, :_D]
    vq_loss = jnp.sum(parts) * ((_COMMIT + 1.0) / z.size)
    return q.reshape(z.shape), idx2.reshape(z.shape[:-1]), vq_loss


# speedup vs baseline: 1.6888x; 1.0689x over previous
import functools

import jax
import jax.numpy as jnp
from jax.experimental import pallas as pl
from jax.experimental.pallas import tpu as pltpu
from jax.experimental.pallas import tpu_sc as plsc

_N = 16384
_T = 2048
_K = 1024
_D = 64
_COMMIT = 0.25

_SC_NC = 2
_SC_NS = 16
_NW = _SC_NC * _SC_NS
_BPW = _N // _NW


def _vq_body(z_ref, cbm2_ref, cbt_ref, idx_ref, part_ref):
    z2d = z_ref[...]
    cbm2 = cbm2_ref[...]
    cbt = cbt_ref[...]
    a = jnp.sum(z2d * z2d, axis=1, keepdims=True)
    b2 = jax.lax.dot_general(z2d, cbm2, (((1,), (1,)), ((), ())),
                             preferred_element_type=jnp.float32)
    c = jnp.sum(cbt * cbt, axis=0, keepdims=True)
    dist = a + b2 + c
    min_d = jnp.min(dist, axis=1, keepdims=True)
    lane_f = jax.lax.broadcasted_iota(jnp.int32, (_T, _K), 1).astype(jnp.float32)
    cand = jnp.where(dist == min_d, lane_f, jnp.float32(_K))
    idx2d = jnp.min(cand, axis=1, keepdims=True).astype(jnp.int32)
    idx_ref[...] = idx2d
    part_ref[0] = jnp.sum(min_d, axis=(0, 1), keepdims=True)


def _tc_argmin(zf, codebook):
    return pl.pallas_call(
        _vq_body,
        grid=(_N // _T,),
        in_specs=[
            pl.BlockSpec((_T, _D), lambda i: (i, 0)),
            pl.BlockSpec((_K, _D), lambda i: (0, 0)),
            pl.BlockSpec((_D, _K), lambda i: (0, 0)),
        ],
        out_specs=[
            pl.BlockSpec((_T, 1), lambda i: (i, 0)),
            pl.BlockSpec((1, 1, 1), lambda i: (i, 0, 0)),
        ],
        out_shape=[
            jax.ShapeDtypeStruct((_N, 1), jnp.int32),
            jax.ShapeDtypeStruct((_N // _T, 1, 1), jnp.float32),
        ],
        compiler_params=pltpu.CompilerParams(
            dimension_semantics=("parallel",)),
    )(zf, codebook * -2.0, codebook.T)


@functools.partial(
    pl.kernel,
    mesh=plsc.VectorSubcoreMesh(core_axis_name="c", subcore_axis_name="s"),
    out_type=jax.ShapeDtypeStruct((_N, 128), jnp.float32),
    scratch_types=[
        pltpu.VMEM((_BPW,), jnp.int32),
        pltpu.VMEM((_BPW, 128), jnp.float32),
        pltpu.SemaphoreType.DMA,
    ],
)
def _sc_gather(table_hbm, idx_hbm, out_hbm, idx_v, rows_v, sem):
    wid = jax.lax.axis_index("s") * _SC_NC + jax.lax.axis_index("c")
    base = wid * _BPW
    pltpu.sync_copy(idx_hbm.at[pl.ds(base, _BPW)], idx_v)
    pltpu.async_copy(table_hbm.at[idx_v], rows_v, sem).wait()
    pltpu.sync_copy(rows_v, out_hbm.at[pl.ds(base, _BPW)])


def kernel(z, codebook):
    zf = z.reshape(_N, _D)
    idx2, parts = _tc_argmin(zf, codebook)
    table_pad = jnp.concatenate(
        [codebook, jnp.zeros((_K, 128 - _D), jnp.float32)], axis=1)
    q = _sc_gather(table_pad, idx2.reshape(_N))[:, :_D]
    vq_loss = jnp.sum(parts) * ((_COMMIT + 1.0) / z.size)
    return q.reshape(z.shape), idx2.reshape(z.shape[:-1]), vq_loss
